# baseline (speedup 1.0000x reference)
import functools

import jax
import jax.numpy as jnp
from jax import lax
from jax.experimental import pallas as pl
from jax.experimental.pallas import tpu as pltpu
from jax.experimental.pallas import tpu_sc as plsc

BATCH = 16384
EMBED_K = 64
NUM_CORES = 2
NUM_SUBCORES = 16
LANES = 16
NUM_WORKERS = NUM_CORES * NUM_SUBCORES

SC_ROWS = 2048
TC_ROWS = BATCH - SC_ROWS
SC_RPW = SC_ROWS // NUM_WORKERS
CHUNK = 64
NUM_CHUNKS = SC_RPW // CHUNK
CHUNK_WORDS = CHUNK * EMBED_K


def _sc_body(gu_hbm, gi_hbm, out_hbm, gu_v, gi_v, out_v, sem_a, sem_b):
    wid = lax.axis_index("s") * NUM_CORES + lax.axis_index("c")
    base = wid * SC_RPW

    sems = (sem_a, sem_b)

    def start(c):
        buf = c % 2
        off = (base + c * CHUNK) * EMBED_K
        h_u = pltpu.make_async_copy(
            gu_hbm.at[pl.ds(off, CHUNK_WORDS)], gu_v.at[buf], sems[buf])
        h_i = pltpu.make_async_copy(
            gi_hbm.at[pl.ds(off, CHUNK_WORDS)], gi_v.at[buf], sems[buf])
        h_u.start()
        h_i.start()
        return h_u, h_i

    last_lane = lax.iota(jnp.int32, LANES) == LANES - 1
    handles = [None, None]
    handles[0] = start(0)
    for c in range(NUM_CHUNKS):
        if c + 1 < NUM_CHUNKS:
            handles[(c + 1) % 2] = start(c + 1)
        h_u, h_i = handles[c % 2]
        h_u.wait()
        h_i.wait()
        buf = c % 2
        row0 = c * CHUNK

        @plsc.parallel_loop(0, CHUNK, unroll=16)
        def _row(r):
            off = r * EMBED_K
            s = gu_v[buf, pl.ds(off, LANES)] * gi_v[buf, pl.ds(off, LANES)]
            for k in range(1, EMBED_K // LANES):
                s = s + (gu_v[buf, pl.ds(off + k * LANES, LANES)]
                         * gi_v[buf, pl.ds(off + k * LANES, LANES)])
            plsc.store_scatter(out_v, [jnp.full((LANES,), row0 + r, jnp.int32)],
                               plsc.cumsum(s), mask=last_lane)

    pltpu.sync_copy(out_v, out_hbm.at[pl.ds(base, SC_RPW)])


_sc_dot = functools.partial(
    pl.kernel,
    mesh=plsc.VectorSubcoreMesh(core_axis_name="c", subcore_axis_name="s",
                                num_cores=NUM_CORES),
    out_type=jax.ShapeDtypeStruct((SC_ROWS,), jnp.float32),
    compiler_params=pltpu.CompilerParams(needs_layout_passes=False),
    scratch_types=[
        pltpu.VMEM((2, CHUNK_WORDS), jnp.float32),
        pltpu.VMEM((2, CHUNK_WORDS), jnp.float32),
        pltpu.VMEM((SC_RPW,), jnp.float32),
        pltpu.SemaphoreType.DMA,
        pltpu.SemaphoreType.DMA,
    ],
)(_sc_body)

TC_BLOCK = 2048


def _tc_body(gu_ref, gi_ref, out_ref):
    out_ref[...] = jnp.sum(gu_ref[...] * gi_ref[...], axis=1)


_tc_dot = pl.pallas_call(
    _tc_body,
    grid=(TC_ROWS // TC_BLOCK,),
    in_specs=[
        pl.BlockSpec((TC_BLOCK, EMBED_K), lambda i: (i, 0)),
        pl.BlockSpec((TC_BLOCK, EMBED_K), lambda i: (i, 0)),
    ],
    out_specs=pl.BlockSpec((TC_BLOCK,), lambda i: (i,)),
    out_shape=jax.ShapeDtypeStruct((TC_ROWS,), jnp.float32),
)


def kernel(gu, gi):
    xui_sc = _sc_dot(gu[:SC_ROWS].reshape(-1), gi[:SC_ROWS].reshape(-1))
    xui_tc = _tc_dot(gu[SC_ROWS:], gi[SC_ROWS:])
    return jnp.concatenate([xui_sc, xui_tc])


# speedup vs baseline: 1.0798x; 1.0798x over previous
import functools

import jax
import jax.numpy as jnp
from jax import lax
from jax.experimental import pallas as pl
from jax.experimental.pallas import tpu as pltpu
from jax.experimental.pallas import tpu_sc as plsc

BATCH = 16384
EMBED_K = 64
NUM_CORES = 2
NUM_SUBCORES = 16
LANES = 16
NUM_WORKERS = NUM_CORES * NUM_SUBCORES

SC_ROWS = 4096
TC_ROWS = BATCH - SC_ROWS
SC_RPW = SC_ROWS // NUM_WORKERS
CHUNK = 64
NUM_CHUNKS = SC_RPW // CHUNK
CHUNK_WORDS = CHUNK * EMBED_K


def _sc_body(gu_hbm, gi_hbm, out_hbm, gu_v, gi_v, out_v, sem_a, sem_b):
    wid = lax.axis_index("s") * NUM_CORES + lax.axis_index("c")
    base = wid * SC_RPW

    sems = (sem_a, sem_b)

    def start(c):
        buf = c % 2
        off = (base + c * CHUNK) * EMBED_K
        h_u = pltpu.make_async_copy(
            gu_hbm.at[pl.ds(off, CHUNK_WORDS)], gu_v.at[buf], sems[buf])
        h_i = pltpu.make_async_copy(
            gi_hbm.at[pl.ds(off, CHUNK_WORDS)], gi_v.at[buf], sems[buf])
        h_u.start()
        h_i.start()
        return h_u, h_i

    last_lane = lax.iota(jnp.int32, LANES) == LANES - 1
    handles = [None, None]
    handles[0] = start(0)
    for c in range(NUM_CHUNKS):
        if c + 1 < NUM_CHUNKS:
            handles[(c + 1) % 2] = start(c + 1)
        h_u, h_i = handles[c % 2]
        h_u.wait()
        h_i.wait()
        buf = c % 2
        row0 = c * CHUNK

        @plsc.parallel_loop(0, CHUNK, unroll=16)
        def _row(r):
            off = r * EMBED_K
            s = gu_v[buf, pl.ds(off, LANES)] * gi_v[buf, pl.ds(off, LANES)]
            for k in range(1, EMBED_K // LANES):
                s = s + (gu_v[buf, pl.ds(off + k * LANES, LANES)]
                         * gi_v[buf, pl.ds(off + k * LANES, LANES)])
            plsc.store_scatter(out_v, [jnp.full((LANES,), row0 + r, jnp.int32)],
                               plsc.cumsum(s), mask=last_lane)

    pltpu.sync_copy(out_v, out_hbm.at[pl.ds(base, SC_RPW)])


_sc_dot = functools.partial(
    pl.kernel,
    mesh=plsc.VectorSubcoreMesh(core_axis_name="c", subcore_axis_name="s",
                                num_cores=NUM_CORES),
    out_type=jax.ShapeDtypeStruct((SC_ROWS,), jnp.float32),
    compiler_params=pltpu.CompilerParams(needs_layout_passes=False),
    scratch_types=[
        pltpu.VMEM((2, CHUNK_WORDS), jnp.float32),
        pltpu.VMEM((2, CHUNK_WORDS), jnp.float32),
        pltpu.VMEM((SC_RPW,), jnp.float32),
        pltpu.SemaphoreType.DMA,
        pltpu.SemaphoreType.DMA,
    ],
)(_sc_body)

TC_BLOCK = 2048


def _tc_body(gu_ref, gi_ref, out_ref):
    out_ref[...] = jnp.sum(gu_ref[...] * gi_ref[...], axis=1)


_tc_dot = pl.pallas_call(
    _tc_body,
    grid=(TC_ROWS // TC_BLOCK,),
    in_specs=[
        pl.BlockSpec((TC_BLOCK, EMBED_K), lambda i: (i, 0)),
        pl.BlockSpec((TC_BLOCK, EMBED_K), lambda i: (i, 0)),
    ],
    out_specs=pl.BlockSpec((TC_BLOCK,), lambda i: (i,)),
    out_shape=jax.ShapeDtypeStruct((TC_ROWS,), jnp.float32),
)


def kernel(gu, gi):
    xui_sc = _sc_dot(gu[:SC_ROWS].reshape(-1), gi[:SC_ROWS].reshape(-1))
    xui_tc = _tc_dot(gu[SC_ROWS:], gi[SC_ROWS:])
    return jnp.concatenate([xui_sc, xui_tc])
